# baseline (speedup 1.0000x reference)
import functools

import jax
import jax.numpy as jnp
from jax import lax
from jax.experimental import pallas as pl
from jax.experimental.pallas import tpu as pltpu
from jax.experimental.pallas import tpu_sc as plsc

N = 1_000_000
M = 500_000
NW = 32
NR = 16
RCHUNK = 62_496
RLAST = N - (NR - 1) * RCHUNK
RBUF = RLAST
MH = M // 2
W = 10_000
NWIN = MH // W
NPAIR = NWIN // 2
WM = 10_000
NMW = RBUF // WM
MTAIL = RBUF - NMW * WM
L = 16


def _make_sc_kernel():
    mesh = plsc.VectorSubcoreMesh(core_axis_name="c", subcore_axis_name="s",
                                  num_cores=2, num_subcores=16)

    @functools.partial(
        pl.kernel,
        out_type=jax.ShapeDtypeStruct((N,), jnp.int32),
        mesh=mesh,
        scratch_types=[
            pltpu.VMEM((W,), jnp.int32),
            pltpu.VMEM((W,), jnp.int32),
            pltpu.VMEM((RBUF,), jnp.int32),
            pltpu.VMEM_SHARED((NR // 2 * RBUF,), jnp.int32),
            pltpu.SemaphoreType.DMA,
            pltpu.SemaphoreType.DMA,
        ],
        compiler_params=pltpu.CompilerParams(needs_layout_passes=False),
    )
    def scatter_kernel(idx_hbm, out_hbm, win0_v, win1_v, out_v, stage_s,
                       sem0, sem1):
        c = lax.axis_index("c")
        s = lax.axis_index("s")
        pair = s >> 1
        rid = pair * 2 + c
        jhalf = s & 1
        base = rid * RCHUNK
        joff = jhalf * MH
        rng = jnp.where(rid == NR - 1, RLAST, RCHUNK).astype(jnp.uint32)
        lane = lax.iota(jnp.int32, L)
        zeros = jnp.zeros((L,), jnp.int32)

        def wait(win_v, sem):
            pltpu.make_async_copy(idx_hbm.at[pl.ds(0, W)], win_v, sem).wait()

        def process(win_v, jbase):
            @plsc.parallel_loop(0, W // L, unroll=32)
            def _vec(i):
                vidx = win_v[pl.ds(i * L, L)]
                loc = vidx - base
                mask = loc.astype(jnp.uint32) < rng
                jv = (jbase + i * L) + lane
                plsc.store_scatter(out_v, [loc], jv, mask=mask)

        pltpu.async_copy(idx_hbm.at[pl.ds(joff, W)], win0_v, sem0)

        @plsc.parallel_loop(0, RBUF // L, unroll=8)
        def _zero(i):
            out_v[pl.ds(i * L, L)] = zeros

        @pl.loop(0, NPAIR)
        def _win(t):
            w0 = 2 * t
            pltpu.async_copy(idx_hbm.at[pl.ds(joff + (w0 + 1) * W, W)],
                             win1_v, sem1)
            wait(win0_v, sem0)
            process(win0_v, joff + w0 * W)
            pltpu.async_copy(idx_hbm.at[pl.ds(joff + (w0 + 2) * W, W)],
                             win0_v, sem0)
            wait(win1_v, sem1)
            process(win1_v, joff + (w0 + 1) * W)

        wait(win0_v, sem0)
        process(win0_v, joff + 2 * NPAIR * W)

        plsc.subcore_barrier()

        slot = pl.multiple_of(pair * RBUF, 8)

        @pl.when(jhalf == 0)
        def _stage():
            pltpu.sync_copy(out_v, stage_s.at[pl.ds(slot, RBUF)])

        plsc.subcore_barrier()

        @pl.when(jhalf == 1)
        def _merge():
            def merge_window(off, nvec, buf):
                @plsc.parallel_loop(0, nvec, unroll=8)
                def _m(i):
                    a = out_v[pl.ds(off + i * L, L)]
                    b = buf[pl.ds(i * L, L)]
                    out_v[pl.ds(off + i * L, L)] = jnp.maximum(a, b)

            @pl.loop(0, NMW)
            def _mw(w):
                off = w * WM
                pltpu.sync_copy(
                    stage_s.at[pl.ds(pl.multiple_of(slot + off, 8), WM)],
                    win0_v)
                merge_window(off, WM // L, win0_v)

            pltpu.sync_copy(
                stage_s.at[pl.ds(pl.multiple_of(slot + NMW * WM, 8), MTAIL)],
                win1_v.at[pl.ds(0, MTAIL)])
            merge_window(NMW * WM, MTAIL // L, win1_v)

            @pl.when(rid == NR - 1)
            def _store_last():
                pltpu.sync_copy(out_v, out_hbm.at[pl.ds(base, RLAST)])

            @pl.when(rid != NR - 1)
            def _store():
                pltpu.sync_copy(out_v.at[pl.ds(0, RCHUNK)],
                                out_hbm.at[pl.ds(base, RCHUNK)])

    return scatter_kernel


_sc_kernel = _make_sc_kernel()


def kernel(X, row_missing_idx):
    del X
    return _sc_kernel(row_missing_idx)


# speedup vs baseline: 1.0498x; 1.0498x over previous
import functools

import jax
import jax.numpy as jnp
from jax import lax
from jax.experimental import pallas as pl
from jax.experimental.pallas import tpu as pltpu
from jax.experimental.pallas import tpu_sc as plsc

N = 1_000_000
M = 500_000
NW = 32
NR = 16
RCHUNK = 62_496
RLAST = N - (NR - 1) * RCHUNK
RBUF = RLAST
MH = M // 2
W = 10_000
NWIN = MH // W
NPAIR = NWIN // 2
WM = 10_000
NMW = RBUF // WM
MTAIL = RBUF - NMW * WM
L = 16


def _make_sc_kernel():
    mesh = plsc.VectorSubcoreMesh(core_axis_name="c", subcore_axis_name="s",
                                  num_cores=2, num_subcores=16)

    @functools.partial(
        pl.kernel,
        out_type=jax.ShapeDtypeStruct((N,), jnp.int32),
        mesh=mesh,
        scratch_types=[
            pltpu.VMEM((W,), jnp.int32),
            pltpu.VMEM((W,), jnp.int32),
            pltpu.VMEM((RBUF,), jnp.int32),
            pltpu.VMEM_SHARED((NR // 2 * RBUF,), jnp.int32),
            pltpu.SemaphoreType.DMA,
            pltpu.SemaphoreType.DMA,
        ],
        compiler_params=pltpu.CompilerParams(needs_layout_passes=False),
    )
    def scatter_kernel(idx_hbm, out_hbm, win0_v, win1_v, out_v, stage_s,
                       sem0, sem1):
        c = lax.axis_index("c")
        s = lax.axis_index("s")
        pair = s >> 1
        rid = pair * 2 + c
        jhalf = s & 1
        base = rid * RCHUNK
        joff = jhalf * MH
        rng = jnp.where(rid == NR - 1, RLAST, RCHUNK).astype(jnp.uint32)
        lane = lax.iota(jnp.int32, L)
        zeros = jnp.zeros((L,), jnp.int32)

        def wait(win_v, sem):
            pltpu.make_async_copy(idx_hbm.at[pl.ds(0, W)], win_v, sem).wait()

        def process(win_v, jbase):
            @plsc.parallel_loop(0, W // L, unroll=16)
            def _vec(i):
                vidx = win_v[pl.ds(i * L, L)]
                loc = vidx - base
                mask = loc.astype(jnp.uint32) < rng
                jv = (jbase + i * L) + lane
                plsc.store_scatter(out_v, [loc], jv, mask=mask)

        pltpu.async_copy(idx_hbm.at[pl.ds(joff, W)], win0_v, sem0)

        @plsc.parallel_loop(0, RBUF // L, unroll=8)
        def _zero(i):
            out_v[pl.ds(i * L, L)] = zeros

        @pl.loop(0, NPAIR)
        def _win(t):
            w0 = 2 * t
            pltpu.async_copy(idx_hbm.at[pl.ds(joff + (w0 + 1) * W, W)],
                             win1_v, sem1)
            wait(win0_v, sem0)
            process(win0_v, joff + w0 * W)
            pltpu.async_copy(idx_hbm.at[pl.ds(joff + (w0 + 2) * W, W)],
                             win0_v, sem0)
            wait(win1_v, sem1)
            process(win1_v, joff + (w0 + 1) * W)

        wait(win0_v, sem0)
        process(win0_v, joff + 2 * NPAIR * W)

        plsc.subcore_barrier()

        slot = pl.multiple_of(pair * RBUF, 8)

        @pl.when(jhalf == 0)
        def _stage():
            pltpu.sync_copy(out_v, stage_s.at[pl.ds(slot, RBUF)])

        plsc.subcore_barrier()

        @pl.when(jhalf == 1)
        def _merge():
            def merge_window(off, nvec, buf):
                @plsc.parallel_loop(0, nvec, unroll=8)
                def _m(i):
                    a = out_v[pl.ds(off + i * L, L)]
                    b = buf[pl.ds(i * L, L)]
                    out_v[pl.ds(off + i * L, L)] = jnp.maximum(a, b)

            bufs = (win0_v, win1_v)
            sems = (sem0, sem1)

            def stage_src(w, size):
                return stage_s.at[pl.ds(pl.multiple_of(slot + w * WM, 8),
                                        size)]

            def msize(w):
                return WM if w < NMW else MTAIL

            pltpu.async_copy(stage_src(0, WM), bufs[0], sems[0])
            for w in range(NMW + 1):
                if w + 1 <= NMW:
                    b = bufs[(w + 1) % 2]
                    pltpu.async_copy(stage_src(w + 1, msize(w + 1)),
                                     b.at[pl.ds(0, msize(w + 1))],
                                     sems[(w + 1) % 2])
                b = bufs[w % 2]
                pltpu.make_async_copy(stage_src(w, msize(w)),
                                      b.at[pl.ds(0, msize(w))],
                                      sems[w % 2]).wait()
                merge_window(w * WM, msize(w) // L, b)

            @pl.when(rid == NR - 1)
            def _store_last():
                pltpu.sync_copy(out_v, out_hbm.at[pl.ds(base, RLAST)])

            @pl.when(rid != NR - 1)
            def _store():
                pltpu.sync_copy(out_v.at[pl.ds(0, RCHUNK)],
                                out_hbm.at[pl.ds(base, RCHUNK)])

    return scatter_kernel


_sc_kernel = _make_sc_kernel()


def kernel(X, row_missing_idx):
    del X
    return _sc_kernel(row_missing_idx)


# speedup vs baseline: 1.1360x; 1.0822x over previous
import functools

import jax
import jax.numpy as jnp
from jax import lax
from jax.experimental import pallas as pl
from jax.experimental.pallas import tpu as pltpu
from jax.experimental.pallas import tpu_sc as plsc

N = 1_000_000
M = 500_000
NW = 32
NR = 16
RCHUNK = 62_496
RLAST = N - (NR - 1) * RCHUNK
RBUF = RLAST
MH = M // 2
W = 10_000
NWIN = MH // W
NPAIR = NWIN // 2
WM = 10_000
NMW = RBUF // WM
MTAIL = RBUF - NMW * WM
L = 16


def _make_sc_kernel():
    mesh = plsc.VectorSubcoreMesh(core_axis_name="c", subcore_axis_name="s",
                                  num_cores=2, num_subcores=16)

    @functools.partial(
        pl.kernel,
        out_type=jax.ShapeDtypeStruct((N,), jnp.int32),
        mesh=mesh,
        scratch_types=[
            pltpu.VMEM((W,), jnp.int32),
            pltpu.VMEM((W,), jnp.int32),
            pltpu.VMEM((W,), jnp.int32),
            pltpu.VMEM((RBUF,), jnp.int32),
            pltpu.VMEM_SHARED((NR // 2 * RBUF,), jnp.int32),
            pltpu.SemaphoreType.DMA,
            pltpu.SemaphoreType.DMA,
            pltpu.SemaphoreType.DMA,
        ],
        compiler_params=pltpu.CompilerParams(needs_layout_passes=False),
    )
    def scatter_kernel(idx_hbm, out_hbm, win0_v, win1_v, win2_v,
                       out_v, stage_s, sem0, sem1, sem2):
        c = lax.axis_index("c")
        s = lax.axis_index("s")
        pair = s >> 1
        rid = pair * 2 + c
        jhalf = s & 1
        base = rid * RCHUNK
        joff = jhalf * MH
        rng = jnp.where(rid == NR - 1, RLAST, RCHUNK).astype(jnp.uint32)
        lane = lax.iota(jnp.int32, L)
        zeros = jnp.zeros((L,), jnp.int32)

        def wait(win_v, sem):
            pltpu.make_async_copy(idx_hbm.at[pl.ds(0, W)], win_v, sem).wait()

        def process(win_v, jbase):
            @plsc.parallel_loop(0, W // L, unroll=16)
            def _vec(i):
                vidx = win_v[pl.ds(i * L, L)]
                loc = vidx - base
                mask = loc.astype(jnp.uint32) < rng
                jv = (jbase + i * L) + lane
                plsc.store_scatter(out_v, [loc], jv, mask=mask)

        wins = (win0_v, win1_v, win2_v)
        sems = (sem0, sem1, sem2)

        def fetch(w, b):
            pltpu.async_copy(idx_hbm.at[pl.ds(joff + w * W, W)],
                             wins[b], sems[b])

        fetch(0, 0)
        fetch(1, 1)

        @plsc.parallel_loop(0, RBUF // L, unroll=8)
        def _zero(i):
            out_v[pl.ds(i * L, L)] = zeros

        @pl.loop(0, NWIN // 3)
        def _win(t):
            k0 = 3 * t
            for b in range(3):
                k = k0 + b

                @pl.when(k + 2 < NWIN)
                def _pf():
                    fetch(k + 2, (b + 2) % 3)

                wait(wins[b], sems[b])
                process(wins[b], joff + k * W)

        wait(wins[(NWIN - 1) % 3], sems[(NWIN - 1) % 3])
        process(wins[(NWIN - 1) % 3], joff + (NWIN - 1) * W)

        plsc.subcore_barrier()

        slot = pl.multiple_of(pair * RBUF, 8)

        @pl.when(jhalf == 0)
        def _stage():
            pltpu.sync_copy(out_v, stage_s.at[pl.ds(slot, RBUF)])

        plsc.subcore_barrier()

        @pl.when(jhalf == 1)
        def _merge():
            def merge_window(off, nvec, buf):
                @plsc.parallel_loop(0, nvec, unroll=8)
                def _m(i):
                    a = out_v[pl.ds(off + i * L, L)]
                    b = buf[pl.ds(i * L, L)]
                    out_v[pl.ds(off + i * L, L)] = jnp.maximum(a, b)

            bufs = (win0_v, win1_v)
            sems = (sem0, sem1)

            def stage_src(w, size):
                return stage_s.at[pl.ds(pl.multiple_of(slot + w * WM, 8),
                                        size)]

            def msize(w):
                return WM if w < NMW else MTAIL

            pltpu.async_copy(stage_src(0, WM), bufs[0], sems[0])
            for w in range(NMW + 1):
                if w + 1 <= NMW:
                    b = bufs[(w + 1) % 2]
                    pltpu.async_copy(stage_src(w + 1, msize(w + 1)),
                                     b.at[pl.ds(0, msize(w + 1))],
                                     sems[(w + 1) % 2])
                b = bufs[w % 2]
                pltpu.make_async_copy(stage_src(w, msize(w)),
                                      b.at[pl.ds(0, msize(w))],
                                      sems[w % 2]).wait()
                merge_window(w * WM, msize(w) // L, b)

            @pl.when(rid == NR - 1)
            def _store_last():
                pltpu.sync_copy(out_v, out_hbm.at[pl.ds(base, RLAST)])

            @pl.when(rid != NR - 1)
            def _store():
                pltpu.sync_copy(out_v.at[pl.ds(0, RCHUNK)],
                                out_hbm.at[pl.ds(base, RCHUNK)])

    return scatter_kernel


_sc_kernel = _make_sc_kernel()


def kernel(X, row_missing_idx):
    del X
    return _sc_kernel(row_missing_idx)


# speedup vs baseline: 1.2440x; 1.0950x over previous
import functools

import jax
import jax.numpy as jnp
from jax import lax
from jax.experimental import pallas as pl
from jax.experimental.pallas import tpu as pltpu
from jax.experimental.pallas import tpu_sc as plsc

N = 1_000_000
M = 500_000
NW = 32
NR = 16
RCHUNK = 62_496
RLAST = N - (NR - 1) * RCHUNK
RBUF = RLAST
MH = M // 2
W = 10_000
NWIN = MH // W
NPAIR = NWIN // 2
HALF = RBUF // 2
WM = 10_000
NMW = HALF // WM
MTAIL = HALF - NMW * WM
L = 16


def _make_sc_kernel():
    mesh = plsc.VectorSubcoreMesh(core_axis_name="c", subcore_axis_name="s",
                                  num_cores=2, num_subcores=16)

    @functools.partial(
        pl.kernel,
        out_type=jax.ShapeDtypeStruct((N,), jnp.int32),
        mesh=mesh,
        scratch_types=[
            pltpu.VMEM((W,), jnp.int32),
            pltpu.VMEM((W,), jnp.int32),
            pltpu.VMEM((W,), jnp.int32),
            pltpu.VMEM((RBUF,), jnp.int32),
            pltpu.VMEM_SHARED((NR // 2 * RBUF,), jnp.int32),
            pltpu.SemaphoreType.DMA,
            pltpu.SemaphoreType.DMA,
            pltpu.SemaphoreType.DMA,
        ],
        compiler_params=pltpu.CompilerParams(needs_layout_passes=False),
    )
    def scatter_kernel(idx_hbm, out_hbm, win0_v, win1_v, win2_v,
                       out_v, stage_s, sem0, sem1, sem2):
        c = lax.axis_index("c")
        s = lax.axis_index("s")
        pair = s >> 1
        rid = pair * 2 + c
        jhalf = s & 1
        base = rid * RCHUNK
        joff = jhalf * MH
        rng = jnp.where(rid == NR - 1, RLAST, RCHUNK).astype(jnp.uint32)
        lane = lax.iota(jnp.int32, L)
        zeros = jnp.zeros((L,), jnp.int32)

        def wait(win_v, sem):
            pltpu.make_async_copy(idx_hbm.at[pl.ds(0, W)], win_v, sem).wait()

        def process(win_v, jbase):
            @plsc.parallel_loop(0, W // L, unroll=16)
            def _vec(i):
                vidx = win_v[pl.ds(i * L, L)]
                loc = vidx - base
                mask = loc.astype(jnp.uint32) < rng
                jv = (jbase + i * L) + lane
                plsc.store_scatter(out_v, [loc], jv, mask=mask)

        wins = (win0_v, win1_v, win2_v)
        sems = (sem0, sem1, sem2)

        def fetch(w, b):
            pltpu.async_copy(idx_hbm.at[pl.ds(joff + w * W, W)],
                             wins[b], sems[b])

        fetch(0, 0)
        fetch(1, 1)

        @plsc.parallel_loop(0, RBUF // L, unroll=8)
        def _zero(i):
            out_v[pl.ds(i * L, L)] = zeros

        @pl.loop(0, NWIN // 3)
        def _win(t):
            k0 = 3 * t
            for b in range(3):
                k = k0 + b

                @pl.when(k + 2 < NWIN)
                def _pf():
                    fetch(k + 2, (b + 2) % 3)

                wait(wins[b], sems[b])
                process(wins[b], joff + k * W)

        wait(wins[(NWIN - 1) % 3], sems[(NWIN - 1) % 3])
        process(wins[(NWIN - 1) % 3], joff + (NWIN - 1) * W)

        plsc.subcore_barrier()

        slot = pl.multiple_of(pair * RBUF, 8)
        my_off = pl.multiple_of(jhalf * HALF, 8)
        st_off = pl.multiple_of((1 - jhalf) * HALF, 8)

        pltpu.sync_copy(out_v.at[pl.ds(st_off, HALF)],
                        stage_s.at[pl.ds(pl.multiple_of(slot + st_off, 8),
                                         HALF)])
        plsc.subcore_barrier()

        def merge_window(off, nvec, buf):
            @plsc.parallel_loop(0, nvec, unroll=8)
            def _m(i):
                a = out_v[pl.ds(off + i * L, L)]
                b = buf[pl.ds(i * L, L)]
                out_v[pl.ds(off + i * L, L)] = jnp.maximum(a, b)

        bufs = (win0_v, win1_v)
        msems = (sem0, sem1)

        def stage_src(w, size):
            return stage_s.at[pl.ds(pl.multiple_of(slot + my_off + w * WM, 8),
                                    size)]

        def msize(w):
            return WM if w < NMW else MTAIL

        pltpu.async_copy(stage_src(0, WM), bufs[0], msems[0])
        for w in range(NMW + 1):
            if w + 1 <= NMW:
                b = bufs[(w + 1) % 2]
                pltpu.async_copy(stage_src(w + 1, msize(w + 1)),
                                 b.at[pl.ds(0, msize(w + 1))],
                                 msems[(w + 1) % 2])
            b = bufs[w % 2]
            pltpu.make_async_copy(stage_src(w, msize(w)),
                                  b.at[pl.ds(0, msize(w))],
                                  msems[w % 2]).wait()
            merge_window(my_off + w * WM, msize(w) // L, b)

        @pl.when(jhalf == 0)
        def _store_lo():
            pltpu.sync_copy(out_v.at[pl.ds(0, HALF)],
                            out_hbm.at[pl.ds(base, HALF)])

        @pl.when(jnp.logical_and(jhalf == 1, rid == NR - 1))
        def _store_hi_last():
            pltpu.sync_copy(out_v.at[pl.ds(HALF, RLAST - HALF)],
                            out_hbm.at[pl.ds(base + HALF, RLAST - HALF)])

        @pl.when(jnp.logical_and(jhalf == 1, rid != NR - 1))
        def _store_hi():
            pltpu.sync_copy(out_v.at[pl.ds(HALF, RCHUNK - HALF)],
                            out_hbm.at[pl.ds(base + HALF, RCHUNK - HALF)])

    return scatter_kernel


_sc_kernel = _make_sc_kernel()


def kernel(X, row_missing_idx):
    del X
    return _sc_kernel(row_missing_idx)


# speedup vs baseline: 1.2443x; 1.0003x over previous
import functools

import jax
import jax.numpy as jnp
from jax import lax
from jax.experimental import pallas as pl
from jax.experimental.pallas import tpu as pltpu
from jax.experimental.pallas import tpu_sc as plsc

N = 1_000_000
M = 500_000
NW = 32
NR = 16
RCHUNK = 62_496
RLAST = N - (NR - 1) * RCHUNK
RBUF = RLAST
MH = M // 2
W = 10_000
NWIN = MH // W
NPAIR = NWIN // 2
HALF = RBUF // 2
WM = 10_000
NMW = HALF // WM
MTAIL = HALF - NMW * WM
L = 16


def _make_sc_kernel():
    mesh = plsc.VectorSubcoreMesh(core_axis_name="c", subcore_axis_name="s",
                                  num_cores=2, num_subcores=16)

    @functools.partial(
        pl.kernel,
        out_type=jax.ShapeDtypeStruct((N,), jnp.int32),
        mesh=mesh,
        scratch_types=[
            pltpu.VMEM((W,), jnp.int32),
            pltpu.VMEM((W,), jnp.int32),
            pltpu.VMEM((W,), jnp.int32),
            pltpu.VMEM((RBUF,), jnp.int32),
            pltpu.VMEM_SHARED((NR // 2 * RBUF,), jnp.int32),
            pltpu.SemaphoreType.DMA,
            pltpu.SemaphoreType.DMA,
            pltpu.SemaphoreType.DMA,
        ],
        compiler_params=pltpu.CompilerParams(needs_layout_passes=False,
                                             disable_bounds_checks=True),
    )
    def scatter_kernel(idx_hbm, out_hbm, win0_v, win1_v, win2_v,
                       out_v, stage_s, sem0, sem1, sem2):
        c = lax.axis_index("c")
        s = lax.axis_index("s")
        pair = s >> 1
        rid = pair * 2 + c
        jhalf = s & 1
        base = rid * RCHUNK
        joff = jhalf * MH
        rng = jnp.where(rid == NR - 1, RLAST, RCHUNK).astype(jnp.uint32)
        lane = lax.iota(jnp.int32, L)
        zeros = jnp.zeros((L,), jnp.int32)

        def wait(win_v, sem):
            pltpu.make_async_copy(idx_hbm.at[pl.ds(0, W)], win_v, sem).wait()

        def process(win_v, jbase):
            @plsc.parallel_loop(0, W // L, unroll=16)
            def _vec(i):
                vidx = win_v[pl.ds(i * L, L)]
                loc = vidx - base
                mask = loc.astype(jnp.uint32) < rng
                jv = (jbase + i * L) + lane
                plsc.store_scatter(out_v, [loc], jv, mask=mask)

        wins = (win0_v, win1_v, win2_v)
        sems = (sem0, sem1, sem2)

        def fetch(w, b):
            pltpu.async_copy(idx_hbm.at[pl.ds(joff + w * W, W)],
                             wins[b], sems[b])

        fetch(0, 0)
        fetch(1, 1)

        @plsc.parallel_loop(0, RBUF // L, unroll=8)
        def _zero(i):
            out_v[pl.ds(i * L, L)] = zeros

        @pl.loop(0, NWIN // 3)
        def _win(t):
            k0 = 3 * t
            for b in range(3):
                k = k0 + b

                @pl.when(k + 2 < NWIN)
                def _pf():
                    fetch(k + 2, (b + 2) % 3)

                wait(wins[b], sems[b])
                process(wins[b], joff + k * W)

        wait(wins[(NWIN - 1) % 3], sems[(NWIN - 1) % 3])
        process(wins[(NWIN - 1) % 3], joff + (NWIN - 1) * W)

        plsc.subcore_barrier()

        slot = pl.multiple_of(pair * RBUF, 8)
        my_off = pl.multiple_of(jhalf * HALF, 8)
        st_off = pl.multiple_of((1 - jhalf) * HALF, 8)

        pltpu.sync_copy(out_v.at[pl.ds(st_off, HALF)],
                        stage_s.at[pl.ds(pl.multiple_of(slot + st_off, 8),
                                         HALF)])
        plsc.subcore_barrier()

        def merge_window(off, nvec, buf):
            @plsc.parallel_loop(0, nvec, unroll=8)
            def _m(i):
                a = out_v[pl.ds(off + i * L, L)]
                b = buf[pl.ds(i * L, L)]
                out_v[pl.ds(off + i * L, L)] = jnp.maximum(a, b)

        bufs = (win0_v, win1_v)
        msems = (sem0, sem1)

        def stage_src(w, size):
            return stage_s.at[pl.ds(pl.multiple_of(slot + my_off + w * WM, 8),
                                    size)]

        def msize(w):
            return WM if w < NMW else MTAIL

        pltpu.async_copy(stage_src(0, WM), bufs[0], msems[0])
        for w in range(NMW + 1):
            if w + 1 <= NMW:
                b = bufs[(w + 1) % 2]
                pltpu.async_copy(stage_src(w + 1, msize(w + 1)),
                                 b.at[pl.ds(0, msize(w + 1))],
                                 msems[(w + 1) % 2])
            b = bufs[w % 2]
            pltpu.make_async_copy(stage_src(w, msize(w)),
                                  b.at[pl.ds(0, msize(w))],
                                  msems[w % 2]).wait()
            merge_window(my_off + w * WM, msize(w) // L, b)

        @pl.when(jhalf == 0)
        def _store_lo():
            pltpu.sync_copy(out_v.at[pl.ds(0, HALF)],
                            out_hbm.at[pl.ds(base, HALF)])

        @pl.when(jnp.logical_and(jhalf == 1, rid == NR - 1))
        def _store_hi_last():
            pltpu.sync_copy(out_v.at[pl.ds(HALF, RLAST - HALF)],
                            out_hbm.at[pl.ds(base + HALF, RLAST - HALF)])

        @pl.when(jnp.logical_and(jhalf == 1, rid != NR - 1))
        def _store_hi():
            pltpu.sync_copy(out_v.at[pl.ds(HALF, RCHUNK - HALF)],
                            out_hbm.at[pl.ds(base + HALF, RCHUNK - HALF)])

    return scatter_kernel


_sc_kernel = _make_sc_kernel()


def kernel(X, row_missing_idx):
    del X
    return _sc_kernel(row_missing_idx)
